# baseline (speedup 1.0000x reference)
import functools

import jax
import jax.numpy as jnp
from jax import lax
from jax.experimental import pallas as pl
from jax.experimental.pallas import tpu as pltpu
from jax.experimental.pallas import tpu_sc as plsc

_B = 16384
_F = 32

_NC, _NS = 2, 16
_NW = _NC * _NS
_BPW = _B // _NW


_NROWS = 1000000


def _sc_gather_body(uid_ref, iid_ref, ugw_ref, igw_ref, umw_ref, imw_ref,
                    gmf_ref, um_ref, im_ref,
                    uidx_v, iidx_v, ufidx, ifidx, ug_v, ig_v, um_v, im_v,
                    sems):
    wid = lax.axis_index("s") * _NC + lax.axis_index("c")
    base = wid * _BPW
    pltpu.sync_copy(uid_ref.at[wid], uidx_v)
    pltpu.sync_copy(iid_ref.at[wid], iidx_v)

    def mk(c, carry):
        coff = c * _NROWS
        for g in range(_BPW // 16):
            sl = pl.ds(g * 16, 16)
            dsl = pl.ds((g % 8) * 16, 16)
            ufidx[c, g // 8, dsl] = uidx_v[sl] + coff
            ifidx[c, g // 8, dsl] = iidx_v[sl] + coff
        return carry

    lax.fori_loop(0, _F, mk, 0)

    def col(c, carry):
        for j in range(_BPW // 128):
            dst = pl.ds(j * 128, 128)
            pltpu.async_copy(ugw_ref.at[ufidx.at[c, j]], ug_v.at[c, dst],
                             sems.at[0])
            pltpu.async_copy(igw_ref.at[ifidx.at[c, j]], ig_v.at[c, dst],
                             sems.at[1])
            pltpu.async_copy(umw_ref.at[ufidx.at[c, j]], um_v.at[c, dst],
                             sems.at[2])
            pltpu.async_copy(imw_ref.at[ifidx.at[c, j]], im_v.at[c, dst],
                             sems.at[3])
        return carry

    lax.fori_loop(0, _F, col, 0)
    bufs = (ug_v, ig_v, um_v, im_v)
    for t in range(4):
        pltpu.make_async_copy(um_ref.at[:, pl.ds(0, _BPW)], bufs[t],
                              sems.at[t]).wait()

    def prod(c, carry):
        for h in range(_BPW // 16):
            sl = pl.ds(h * 16, 16)
            ug_v[c, sl] = ug_v[c, sl] * ig_v[c, sl]
        return carry

    lax.fori_loop(0, _F, prod, 0)
    cols = pl.ds(base, _BPW)
    pltpu.sync_copy(ug_v, gmf_ref.at[:, cols])
    pltpu.sync_copy(um_v, um_ref.at[:, cols])
    pltpu.sync_copy(im_v, im_ref.at[:, cols])


@functools.cache
def _build_sc_gather():
  return pl.kernel(
    _sc_gather_body,
    out_type=(
        jax.ShapeDtypeStruct((_F, _B), jnp.float32),
        jax.ShapeDtypeStruct((_F, _B), jnp.float32),
        jax.ShapeDtypeStruct((_F, _B), jnp.float32),
    ),
    mesh=plsc.VectorSubcoreMesh(core_axis_name="c", subcore_axis_name="s"),
    compiler_params=pltpu.CompilerParams(use_tc_tiling_on_sc=False),
    scratch_types=[
        pltpu.VMEM((_BPW,), jnp.int32),
        pltpu.VMEM((_BPW,), jnp.int32),
        pltpu.VMEM((_F, _BPW // 128, 128), jnp.int32),
        pltpu.VMEM((_F, _BPW // 128, 128), jnp.int32),
        pltpu.VMEM((_F, _BPW), jnp.float32),
        pltpu.VMEM((_F, _BPW), jnp.float32),
        pltpu.VMEM((_F, _BPW), jnp.float32),
        pltpu.VMEM((_F, _BPW), jnp.float32),
        pltpu.SemaphoreType.DMA((4,)),
    ],
  )


def _tc_mlp_body(gmf_ref, um_ref, im_ref, w1u_ref, w1i_ref, b1_ref,
                 w2_ref, b2_ref, w3_ref, b3_ref, wog_ref, woh_ref, bo_ref,
                 out_ref):
    f32 = jnp.float32
    h = jnp.dot(w1u_ref[:], um_ref[:], preferred_element_type=f32)
    h = h + jnp.dot(w1i_ref[:], im_ref[:], preferred_element_type=f32)
    h = jnp.maximum(h + b1_ref[:], 0.0)
    h = jnp.maximum(
        jnp.dot(w2_ref[:], h, preferred_element_type=f32) + b2_ref[:], 0.0)
    h = jnp.maximum(
        jnp.dot(w3_ref[:], h, preferred_element_type=f32) + b3_ref[:], 0.0)
    logit = jnp.dot(wog_ref[:], gmf_ref[:], preferred_element_type=f32)
    logit = logit + jnp.dot(woh_ref[:], h, preferred_element_type=f32)
    logit = logit + bo_ref[:]
    out_ref[:] = jax.nn.sigmoid(logit)


_TC_BLOCK = 2048
_TC_GRID = _B // _TC_BLOCK


def _full(shape):
    return pl.BlockSpec(shape, lambda i: (0,) * len(shape))


_tc_mlp = pl.pallas_call(
    _tc_mlp_body,
    grid=(_TC_GRID,),
    in_specs=[
        pl.BlockSpec((_F, _TC_BLOCK), lambda i: (0, i)),
        pl.BlockSpec((_F, _TC_BLOCK), lambda i: (0, i)),
        pl.BlockSpec((_F, _TC_BLOCK), lambda i: (0, i)),
        _full((64, _F)), _full((64, _F)), _full((64, 1)),
        _full((32, 64)), _full((32, 1)),
        _full((16, 32)), _full((16, 1)),
        _full((1, _F)), _full((1, 16)), _full((1, 1)),
    ],
    out_specs=pl.BlockSpec((1, _TC_BLOCK), lambda i: (0, i)),
    out_shape=jax.ShapeDtypeStruct((1, _B), jnp.float32),
    compiler_params=pltpu.CompilerParams(
        dimension_semantics=("arbitrary",)),
)


@jax.jit
def kernel(user_id, item_id, user_gmf_w, item_gmf_w, user_mlp_w, item_mlp_w,
           W1, b1, W2, b2, W3, b3, Wo, bo):
    uid2 = user_id.astype(jnp.int32).reshape(_NW, _BPW)
    iid2 = item_id.astype(jnp.int32).reshape(_NW, _BPW)
    gmf, um, im = _build_sc_gather()(
        uid2, iid2,
        user_gmf_w.T.reshape(-1), item_gmf_w.T.reshape(-1),
        user_mlp_w.T.reshape(-1), item_mlp_w.T.reshape(-1))
    out = _tc_mlp(gmf, um, im,
                  W1[:_F].T, W1[_F:].T, b1.reshape(64, 1),
                  W2.T, b2.reshape(32, 1),
                  W3.T, b3.reshape(16, 1),
                  Wo[:_F].T, Wo[_F:].T, bo.reshape(1, 1))
    return jnp.squeeze(out, axis=0)
